# baseline (speedup 1.0000x reference)
import functools
import math

import jax
import jax.numpy as jnp
from jax import lax
from jax.experimental import pallas as pl
from jax.experimental.pallas import tpu as pltpu
from jax.experimental.pallas import tpu_sc as plsc

D_MODEL = 64
SCALE = math.sqrt(D_MODEL)

NC = 2
NS = 16
NW = NC * NS

CHUNK = 1024
IDX_W = 128
IDX_ROWS = CHUNK // IDX_W


def _emb_body(x_hbm, lut_hbm, out_hbm, idx_v, rows_v, sem, *, b_per_w):
    wid = lax.axis_index("s") * NC + lax.axis_index("c")
    n_chunks = b_per_w // CHUNK
    base_idx_row = wid * (b_per_w // IDX_W)
    base_out = wid * b_per_w

    def chunk_body(g, carry):
        pltpu.sync_copy(x_hbm.at[pl.ds(base_idx_row + g * IDX_ROWS, IDX_ROWS)],
                        idx_v)
        copies = [
            pltpu.async_copy(lut_hbm.at[idx_v.at[j]],
                             rows_v.at[pl.ds(j * IDX_W, IDX_W)], sem)
            for j in range(IDX_ROWS)
        ]
        for c in copies:
            c.wait()

        def row_body(r, rc):
            for k in range(D_MODEL // 16):
                rows_v[r, pl.ds(k * 16, 16)] = (
                    rows_v[r, pl.ds(k * 16, 16)] * SCALE)
            return rc
        lax.fori_loop(0, CHUNK, row_body, 0, unroll=2)

        pltpu.sync_copy(rows_v, out_hbm.at[pl.ds(base_out + g * CHUNK, CHUNK)])
        return carry

    lax.fori_loop(0, n_chunks, chunk_body, 0)


@jax.jit
def _run(x2, lut):
    n_idx_rows, _ = x2.shape
    b_total = n_idx_rows * IDX_W
    b_per_w = b_total // NW
    mesh = plsc.VectorSubcoreMesh(core_axis_name="c", subcore_axis_name="s",
                                  num_cores=NC, num_subcores=NS)
    f = pl.kernel(
        functools.partial(_emb_body, b_per_w=b_per_w),
        out_type=jax.ShapeDtypeStruct((b_total, D_MODEL), jnp.float32),
        mesh=mesh,
        scratch_types=[
            pltpu.VMEM((IDX_ROWS, IDX_W), jnp.int32),
            pltpu.VMEM((CHUNK, D_MODEL), jnp.float32),
            pltpu.SemaphoreType.DMA,
        ],
        compiler_params=pltpu.CompilerParams(use_tc_tiling_on_sc=False),
    )
    return f(x2, lut)


def kernel(x, lut):
    b_total = x.shape[0] * x.shape[1]
    assert b_total % (NW * CHUNK) == 0
    out = _run(x.reshape(-1, IDX_W), lut)
    return out.reshape(x.shape + (D_MODEL,))


# speedup vs baseline: 1.0640x; 1.0640x over previous
import math

import jax
import jax.numpy as jnp
from jax import lax
from jax.experimental import pallas as pl
from jax.experimental.pallas import tpu as pltpu
from jax.experimental.pallas import tpu_sc as plsc

D_MODEL = 64
SCALE = math.sqrt(D_MODEL)

NC = 2
NS = 16
NW = NC * NS

CHUNK = 128
NBUF = 8
LOOK = 4
DRAIN = NBUF - LOOK
GROUP = NBUF


def _emb_body(x_hbm, lut_hbm, out_hbm, idx_v, rows_v, gsem, ssem, *, b_per_w):
    wid = lax.axis_index("s") * NC + lax.axis_index("c")
    n_groups = b_per_w // (GROUP * CHUNK)
    idx_row_base = wid * (b_per_w // CHUNK)
    out_base = wid * b_per_w

    def load_idx(k):
        pltpu.sync_copy(x_hbm.at[pl.ds(idx_row_base + k * GROUP, GROUP)],
                        idx_v.at[k % 2])

    def fire_gather(slot, row, b):
        pltpu.async_copy(lut_hbm.at[idx_v.at[slot, row]], rows_v.at[b], gsem)

    def wait_gather(b):
        pltpu.make_async_copy(lut_hbm.at[idx_v.at[0, 0]], rows_v.at[b],
                              gsem).wait()

    def scale(b):
        @plsc.parallel_loop(0, CHUNK, 1, unroll=4)
        def _(r):
            for k in range(D_MODEL // 16):
                rows_v[b, r, pl.ds(k * 16, 16)] = (
                    rows_v[b, r, pl.ds(k * 16, 16)] * SCALE)

    def fire_scatter(s, b):
        pltpu.async_copy(rows_v.at[b],
                         out_hbm.at[pl.ds(out_base + s * CHUNK, CHUNK)], ssem)

    def wait_scatter():
        pltpu.make_async_copy(rows_v.at[0],
                              out_hbm.at[pl.ds(out_base, CHUNK)], ssem).wait()

    load_idx(0)
    for b in range(LOOK):
        fire_gather(0, b, b)

    def group_body(g, *, first, last):
        for b in range(GROUP):
            s = g * GROUP + b
            if b == LOOK and not last:
                load_idx(g + 1)
            wait_gather(b)
            scale(b)
            fire_scatter(s, b)
            if not (first and b < DRAIN):
                wait_scatter()
            if not (last and b >= GROUP - LOOK):
                slot = (g + (1 if b >= GROUP - LOOK else 0)) % 2
                row = (b + LOOK) % GROUP
                fire_gather(slot, row, (b + LOOK) % NBUF)

    group_body(0, first=True, last=False)

    def mid(g, carry):
        group_body(g, first=False, last=False)
        return carry
    lax.fori_loop(1, n_groups - 1, mid, 0)

    group_body(n_groups - 1, first=False, last=True)

    for _ in range(DRAIN):
        wait_scatter()


@jax.jit
def _run(x2, lut):
    n_idx_rows, _ = x2.shape
    b_total = n_idx_rows * CHUNK
    b_per_w = b_total // NW
    mesh = plsc.VectorSubcoreMesh(core_axis_name="c", subcore_axis_name="s",
                                  num_cores=NC, num_subcores=NS)

    def body(x_ref, lut_ref, out_ref, idx_v, rows_v, gsem, ssem):
        _emb_body(x_ref, lut_ref, out_ref, idx_v, rows_v, gsem, ssem,
                  b_per_w=b_per_w)

    f = pl.kernel(
        body,
        out_type=jax.ShapeDtypeStruct((b_total, D_MODEL), jnp.float32),
        mesh=mesh,
        scratch_types=[
            pltpu.VMEM((2, GROUP, CHUNK), jnp.int32),
            pltpu.VMEM((NBUF, CHUNK, D_MODEL), jnp.float32),
            pltpu.SemaphoreType.DMA,
            pltpu.SemaphoreType.DMA,
        ],
        compiler_params=pltpu.CompilerParams(use_tc_tiling_on_sc=False),
    )
    return f(x2, lut)


def kernel(x, lut):
    b_total = x.shape[0] * x.shape[1]
    assert b_total % (NW * GROUP * CHUNK) == 0
    out = _run(x.reshape(-1, CHUNK), lut)
    return out.reshape(x.shape + (D_MODEL,))
